# baseline (speedup 1.0000x reference)
import jax
import jax.numpy as jnp
from jax import lax
from jax.experimental import pallas as pl
from jax.experimental.pallas import tpu as pltpu
from jax.experimental.pallas import tpu_sc as plsc

B, L, V = 4096, 20, 100000
N = B * L
D = 512
SFDIM = 113

G = 8
NG = N // G

NC, NS = 2, 16
NW = NC * NS
PER_W = NG // NW
CHUNK = 32
NBODY = PER_W // CHUNK // 2


def _gather_body(vid_ref, aid_ref, vtab_ref, atab_ref, gv_ref, ga_ref,
                 ia0, ib0, ia1, ib1, ra0, rb0, ra1, rb1,
                 sga0, sgb0, sga1, sgb1, soa0, sob0, soa1, sob1):
    wid = lax.axis_index("s") * NC + lax.axis_index("c")
    base = wid * PER_W

    slots = (
        (vid_ref, vtab_ref, gv_ref, ia0, ra0, sga0, soa0, 0),
        (aid_ref, atab_ref, ga_ref, ib0, rb0, sgb0, sob0, 0),
        (vid_ref, vtab_ref, gv_ref, ia1, ra1, sga1, soa1, 1),
        (aid_ref, atab_ref, ga_ref, ib1, rb1, sgb1, sob1, 1),
    )

    def body(cc, carry):
        gathers = []
        for (src_idx, table, dst, idx, rows, sg, so, par) in slots:
            off = base + (2 * cc + par) * CHUNK

            @pl.when(cc > 0)
            def _():
                pltpu.make_async_copy(dst.at[pl.ds(0, CHUNK)], rows,
                                      so).wait()

            pltpu.sync_copy(src_idx.at[pl.ds(off, CHUNK)], idx)
            gathers.append(pltpu.async_copy(table.at[idx], rows, sg))
        for (src_idx, table, dst, idx, rows, sg, so, par), g in zip(slots,
                                                                    gathers):
            off = base + (2 * cc + par) * CHUNK
            g.wait()
            pltpu.async_copy(rows, dst.at[pl.ds(off, CHUNK)], so)
        return carry

    lax.fori_loop(0, NBODY, body, 0)
    for (src_idx, table, dst, idx, rows, sg, so, par) in slots:
        pltpu.make_async_copy(dst.at[pl.ds(0, CHUNK)], rows, so).wait()


def _sc_gather(vid_flat, aid_flat, vid_table, aid_table):
    mesh = plsc.VectorSubcoreMesh(core_axis_name="c", subcore_axis_name="s",
                                  num_cores=NC, num_subcores=NS)
    return pl.kernel(
        _gather_body,
        out_type=(jax.ShapeDtypeStruct((NG, D), jnp.float32),
                  jax.ShapeDtypeStruct((NG, D), jnp.float32)),
        mesh=mesh,
        scratch_types=[
            pltpu.VMEM((CHUNK,), jnp.int32),
            pltpu.VMEM((CHUNK,), jnp.int32),
            pltpu.VMEM((CHUNK,), jnp.int32),
            pltpu.VMEM((CHUNK,), jnp.int32),
            pltpu.VMEM((CHUNK, D), jnp.float32),
            pltpu.VMEM((CHUNK, D), jnp.float32),
            pltpu.VMEM((CHUNK, D), jnp.float32),
            pltpu.VMEM((CHUNK, D), jnp.float32),
            pltpu.SemaphoreType.DMA,
            pltpu.SemaphoreType.DMA,
            pltpu.SemaphoreType.DMA,
            pltpu.SemaphoreType.DMA,
            pltpu.SemaphoreType.DMA,
            pltpu.SemaphoreType.DMA,
            pltpu.SemaphoreType.DMA,
            pltpu.SemaphoreType.DMA,
        ],
    )(vid_flat, aid_flat, vid_table, aid_table)


def _fold_body(tagW, tsW, ptW, durW, labelW, tagb, tsb, ptb, durb, labelb,
               p1W, p1b, wsf_ref, bias_ref):
    Wt = p1W[1024:1152, :]
    Wts = p1W[1152:1280, :]
    Wp = p1W[1280:1408, :]
    Wd = p1W[1408:1536, :]
    Wl = p1W[1536:1664, :]
    f32 = jnp.float32
    wsf_ref[0:100, :] = jnp.dot(tagW[...], Wt, preferred_element_type=f32)
    wsf_ref[100:101, :] = jnp.dot(tsW[...], Wts, preferred_element_type=f32)
    wsf_ref[101:102, :] = jnp.dot(ptW[...], Wp, preferred_element_type=f32)
    wsf_ref[102:103, :] = jnp.dot(durW[...], Wd, preferred_element_type=f32)
    wsf_ref[103:113, :] = jnp.dot(labelW[...], Wl, preferred_element_type=f32)
    bias_ref[...] = (
        jnp.dot(tagb[...], Wt, preferred_element_type=f32)
        + jnp.dot(tsb[...], Wts, preferred_element_type=f32)
        + jnp.dot(ptb[...], Wp, preferred_element_type=f32)
        + jnp.dot(durb[...], Wd, preferred_element_type=f32)
        + jnp.dot(labelb[...], Wl, preferred_element_type=f32)
        + p1b[...]
    )


def _fold(tag_W, ts_W, pt_W, dur_W, label_W, tag_b, ts_b, pt_b, dur_b,
          label_b, p1_W, p1_b):
    row = lambda b: b.reshape(1, -1)
    return pl.pallas_call(
        _fold_body,
        out_shape=(jax.ShapeDtypeStruct((SFDIM, D), jnp.float32),
                   jax.ShapeDtypeStruct((1, D), jnp.float32)),
    )(tag_W, ts_W, pt_W, dur_W, label_W, row(tag_b), row(ts_b), row(pt_b),
      row(dur_b), row(label_b), p1_W, row(p1_b))


BB = 128
NB = B // BB
RB = BB * L

_DNT = (((0,), (0,)), ((), ()))


def _mlp_body(gv, ga, tag_r, lab_r, ts_r, pt_r, dur_r, Wv, Wa, Wsf, b1,
              p2W, p2b, out, h_scr):
    f32, bf16 = jnp.float32, jnp.bfloat16
    bigh = jnp.dot(gv[...].astype(bf16), Wv[...].astype(bf16),
                   preferred_element_type=f32)
    bigh = bigh + jnp.dot(ga[...].astype(bf16), Wa[...].astype(bf16),
                          preferred_element_type=f32)
    wsf_b = Wsf[...].astype(bf16)
    tag_b16 = tag_r[...].astype(bf16)
    lab_b16 = lab_r[...].astype(bf16)
    for l in range(L):
        hl = bigh[l * BB:(l + 1) * BB, :]
        hl = hl + lax.dot_general(tag_b16[l], wsf_b[0:100, :], _DNT,
                                  preferred_element_type=f32)
        hl = hl + lax.dot_general(lab_b16[:, l, :], wsf_b[103:113, :], _DNT,
                                  preferred_element_type=f32)
        s3 = jnp.concatenate([ts_r[l:l + 1, :], pt_r[l:l + 1, :],
                              dur_r[l:l + 1, :]], axis=0)
        hl = hl + lax.dot_general(s3, Wsf[100:103, :], _DNT,
                                  preferred_element_type=f32)
        hl = hl + b1[...]
        hl = jnp.where(hl >= 0, hl, 0.01 * hl)
        h_scr[l * BB:(l + 1) * BB, :] = hl.astype(bf16)
    out2 = jnp.dot(h_scr[...], p2W[...].astype(bf16),
                   preferred_element_type=f32)
    out2 = out2 + p2b[...]
    for l in range(L):
        out[l, :, :] = out2[l * BB:(l + 1) * BB, :]


def _mlp_body_aliased(prev, *args):
    del prev
    _mlp_body(*args)


NBG = NB // G


def _mlp_group(g, out_prev, gv, ga, tag, label, ts, playtime, dur, p1_W,
               wsf, b1, p2_W, p2_b):
    const = lambda shape: pl.BlockSpec(shape, lambda i: (0, 0))
    o = g * NBG
    in_specs = [
        pl.BlockSpec((RB, D), lambda i: (i, 0)),
        pl.BlockSpec((RB, D), lambda i: (i, 0)),
        pl.BlockSpec((L, 100, BB), lambda i: (0, 0, o + i)),
        pl.BlockSpec((10, L, BB), lambda i: (0, 0, o + i)),
        pl.BlockSpec((L, BB), lambda i: (0, o + i)),
        pl.BlockSpec((L, BB), lambda i: (0, o + i)),
        pl.BlockSpec((L, BB), lambda i: (0, o + i)),
        pl.BlockSpec((D, D), lambda i: (0, 0)),
        pl.BlockSpec((D, D), lambda i: (1, 0)),
        const((SFDIM, D)),
        const((1, D)),
        const((D, D)),
        const((1, D)),
    ]
    args = (gv, ga, tag, label, ts, playtime, dur, p1_W, p1_W, wsf, b1,
            p2_W, p2_b.reshape(1, D))
    body = _mlp_body
    aliases = {}
    if out_prev is not None:
        in_specs = [pl.BlockSpec(memory_space=pl.ANY)] + in_specs
        args = (out_prev,) + args
        body = _mlp_body_aliased
        aliases = {0: 0}
    return pl.pallas_call(
        body,
        grid=(NBG,),
        in_specs=in_specs,
        out_specs=pl.BlockSpec((L, BB, D), lambda i: (0, o + i, 0)),
        out_shape=jax.ShapeDtypeStruct((L, B, D), jnp.float32),
        scratch_shapes=[pltpu.VMEM((RB, D), jnp.bfloat16)],
        input_output_aliases=aliases,
    )(*args)


def _permute_idx(x):
    return (jnp.swapaxes(x.reshape(NB, BB, L), 1, 2)
            .reshape(N).astype(jnp.int32))


def kernel(vid, aid, tag, ts, playtime, dur, label, vid_table, aid_table,
           tag_W, tag_b, ts_W, ts_b, pt_W, pt_b, dur_W, dur_b, label_W,
           label_b, p1_W, p1_b, p2_W, p2_b):
    vid_p = _permute_idx(vid)
    aid_p = _permute_idx(aid)
    wsf, b1 = _fold(tag_W, ts_W, pt_W, dur_W, label_W, tag_b, ts_b, pt_b,
                    dur_b, label_b, p1_W, p1_b)
    tag_t = jnp.transpose(tag, (1, 2, 0))
    lab_t = jnp.transpose(label, (2, 1, 0))
    ts_t = ts.T
    pt_t = playtime.T
    dur_t = dur.T
    gathered = [
        _sc_gather(lax.slice(vid_p, (g * NG,), ((g + 1) * NG,)),
                   lax.slice(aid_p, (g * NG,), ((g + 1) * NG,)),
                   vid_table, aid_table)
        for g in range(G)
    ]
    out = None
    for g in range(G):
        gv, ga = gathered[g]
        out = _mlp_group(g, out, gv, ga, tag_t, lab_t, ts_t, pt_t, dur_t,
                         p1_W, wsf, b1, p2_W, p2_b)
    return jnp.transpose(out, (1, 0, 2))


# speedup vs baseline: 1.0173x; 1.0173x over previous
import jax
import jax.numpy as jnp
from jax import lax
from jax.experimental import pallas as pl
from jax.experimental.pallas import tpu as pltpu
from jax.experimental.pallas import tpu_sc as plsc

B, L, V = 4096, 20, 100000
N = B * L
D = 512
SFDIM = 113

GSIZES = (4, 12, 12, 4)
G = len(GSIZES)
GSTART = tuple(sum(GSIZES[:g]) for g in range(G))

NC, NS = 2, 16
NW = NC * NS
CHUNK = 32


def _make_gather_body(per_w, nbody):
    def _gather_body(vid_ref, aid_ref, vtab_ref, atab_ref, gv_ref, ga_ref,
                     ia0, ib0, ia1, ib1, ra0, rb0, ra1, rb1,
                     sga0, sgb0, sga1, sgb1, soa0, sob0, soa1, sob1):
        wid = lax.axis_index("s") * NC + lax.axis_index("c")
        base = wid * per_w

        slots = (
            (vid_ref, vtab_ref, gv_ref, ia0, ra0, sga0, soa0, 0),
            (aid_ref, atab_ref, ga_ref, ib0, rb0, sgb0, sob0, 0),
            (vid_ref, vtab_ref, gv_ref, ia1, ra1, sga1, soa1, 1),
            (aid_ref, atab_ref, ga_ref, ib1, rb1, sgb1, sob1, 1),
        )

        def body(cc, carry):
            gathers = []
            for (src_idx, table, dst, idx, rows, sg, so, par) in slots:
                off = base + (2 * cc + par) * CHUNK

                @pl.when(cc > 0)
                def _():
                    pltpu.make_async_copy(dst.at[pl.ds(0, CHUNK)], rows,
                                          so).wait()

                pltpu.sync_copy(src_idx.at[pl.ds(off, CHUNK)], idx)
                gathers.append(pltpu.async_copy(table.at[idx], rows, sg))
            for (src_idx, table, dst, idx, rows, sg, so, par), g in zip(
                    slots, gathers):
                off = base + (2 * cc + par) * CHUNK
                g.wait()
                pltpu.async_copy(rows, dst.at[pl.ds(off, CHUNK)], so)
            return carry

        lax.fori_loop(0, nbody, body, 0)
        for (src_idx, table, dst, idx, rows, sg, so, par) in slots:
            pltpu.make_async_copy(dst.at[pl.ds(0, CHUNK)], rows, so).wait()

    return _gather_body


def _sc_gather(vid_flat, aid_flat, vid_table, aid_table):
    ng = vid_flat.shape[0]
    per_w = ng // NW
    mesh = plsc.VectorSubcoreMesh(core_axis_name="c", subcore_axis_name="s",
                                  num_cores=NC, num_subcores=NS)
    return pl.kernel(
        _make_gather_body(per_w, per_w // CHUNK // 2),
        out_type=(jax.ShapeDtypeStruct((ng, D), jnp.float32),
                  jax.ShapeDtypeStruct((ng, D), jnp.float32)),
        mesh=mesh,
        scratch_types=[
            pltpu.VMEM((CHUNK,), jnp.int32),
            pltpu.VMEM((CHUNK,), jnp.int32),
            pltpu.VMEM((CHUNK,), jnp.int32),
            pltpu.VMEM((CHUNK,), jnp.int32),
            pltpu.VMEM((CHUNK, D), jnp.float32),
            pltpu.VMEM((CHUNK, D), jnp.float32),
            pltpu.VMEM((CHUNK, D), jnp.float32),
            pltpu.VMEM((CHUNK, D), jnp.float32),
            pltpu.SemaphoreType.DMA,
            pltpu.SemaphoreType.DMA,
            pltpu.SemaphoreType.DMA,
            pltpu.SemaphoreType.DMA,
            pltpu.SemaphoreType.DMA,
            pltpu.SemaphoreType.DMA,
            pltpu.SemaphoreType.DMA,
            pltpu.SemaphoreType.DMA,
        ],
    )(vid_flat, aid_flat, vid_table, aid_table)


def _fold_body(tagW, tsW, ptW, durW, labelW, tagb, tsb, ptb, durb, labelb,
               p1W, p1b, wsf_ref, bias_ref):
    Wt = p1W[1024:1152, :]
    Wts = p1W[1152:1280, :]
    Wp = p1W[1280:1408, :]
    Wd = p1W[1408:1536, :]
    Wl = p1W[1536:1664, :]
    f32 = jnp.float32
    wsf_ref[0:100, :] = jnp.dot(tagW[...], Wt, preferred_element_type=f32)
    wsf_ref[100:101, :] = jnp.dot(tsW[...], Wts, preferred_element_type=f32)
    wsf_ref[101:102, :] = jnp.dot(ptW[...], Wp, preferred_element_type=f32)
    wsf_ref[102:103, :] = jnp.dot(durW[...], Wd, preferred_element_type=f32)
    wsf_ref[103:113, :] = jnp.dot(labelW[...], Wl, preferred_element_type=f32)
    bias_ref[...] = (
        jnp.dot(tagb[...], Wt, preferred_element_type=f32)
        + jnp.dot(tsb[...], Wts, preferred_element_type=f32)
        + jnp.dot(ptb[...], Wp, preferred_element_type=f32)
        + jnp.dot(durb[...], Wd, preferred_element_type=f32)
        + jnp.dot(labelb[...], Wl, preferred_element_type=f32)
        + p1b[...]
    )


def _fold(tag_W, ts_W, pt_W, dur_W, label_W, tag_b, ts_b, pt_b, dur_b,
          label_b, p1_W, p1_b):
    row = lambda b: b.reshape(1, -1)
    return pl.pallas_call(
        _fold_body,
        out_shape=(jax.ShapeDtypeStruct((SFDIM, D), jnp.float32),
                   jax.ShapeDtypeStruct((1, D), jnp.float32)),
    )(tag_W, ts_W, pt_W, dur_W, label_W, row(tag_b), row(ts_b), row(pt_b),
      row(dur_b), row(label_b), p1_W, row(p1_b))


BB = 128
NB = B // BB
RB = BB * L

_DNT = (((0,), (0,)), ((), ()))


def _mlp_body(gv, ga, tag_r, lab_r, ts_r, pt_r, dur_r, Wv, Wa, Wsf, b1,
              p2W, p2b, out, h_scr):
    f32, bf16 = jnp.float32, jnp.bfloat16
    bigh = jnp.dot(gv[...].astype(bf16), Wv[...].astype(bf16),
                   preferred_element_type=f32)
    bigh = bigh + jnp.dot(ga[...].astype(bf16), Wa[...].astype(bf16),
                          preferred_element_type=f32)
    wsf_b = Wsf[...].astype(bf16)
    tag_b16 = tag_r[...].astype(bf16)
    lab_b16 = lab_r[...].astype(bf16)
    for l in range(L):
        hl = bigh[l * BB:(l + 1) * BB, :]
        hl = hl + lax.dot_general(tag_b16[l], wsf_b[0:100, :], _DNT,
                                  preferred_element_type=f32)
        hl = hl + lax.dot_general(lab_b16[:, l, :], wsf_b[103:113, :], _DNT,
                                  preferred_element_type=f32)
        s3 = jnp.concatenate([ts_r[l:l + 1, :], pt_r[l:l + 1, :],
                              dur_r[l:l + 1, :]], axis=0)
        hl = hl + lax.dot_general(s3, Wsf[100:103, :], _DNT,
                                  preferred_element_type=f32)
        hl = hl + b1[...]
        hl = jnp.where(hl >= 0, hl, 0.01 * hl)
        h_scr[l * BB:(l + 1) * BB, :] = hl.astype(bf16)
    out2 = jnp.dot(h_scr[...], p2W[...].astype(bf16),
                   preferred_element_type=f32)
    out2 = out2 + p2b[...]
    for l in range(L):
        out[l, :, :] = out2[l * BB:(l + 1) * BB, :]


def _mlp_body_aliased(prev, *args):
    del prev
    _mlp_body(*args)


def _mlp_group(g, out_prev, gv, ga, tag, label, ts, playtime, dur, p1_W,
               wsf, b1, p2_W, p2_b):
    const = lambda shape: pl.BlockSpec(shape, lambda i: (0, 0))
    o = GSTART[g]
    in_specs = [
        pl.BlockSpec((RB, D), lambda i: (i, 0)),
        pl.BlockSpec((RB, D), lambda i: (i, 0)),
        pl.BlockSpec((L, 100, BB), lambda i: (0, 0, o + i)),
        pl.BlockSpec((10, L, BB), lambda i: (0, 0, o + i)),
        pl.BlockSpec((L, BB), lambda i: (0, o + i)),
        pl.BlockSpec((L, BB), lambda i: (0, o + i)),
        pl.BlockSpec((L, BB), lambda i: (0, o + i)),
        pl.BlockSpec((D, D), lambda i: (0, 0)),
        pl.BlockSpec((D, D), lambda i: (1, 0)),
        const((SFDIM, D)),
        const((1, D)),
        const((D, D)),
        const((1, D)),
    ]
    args = (gv, ga, tag, label, ts, playtime, dur, p1_W, p1_W, wsf, b1,
            p2_W, p2_b.reshape(1, D))
    body = _mlp_body
    aliases = {}
    if out_prev is not None:
        in_specs = [pl.BlockSpec(memory_space=pl.ANY)] + in_specs
        args = (out_prev,) + args
        body = _mlp_body_aliased
        aliases = {0: 0}
    return pl.pallas_call(
        body,
        grid=(GSIZES[g],),
        in_specs=in_specs,
        out_specs=pl.BlockSpec((L, BB, D), lambda i: (0, o + i, 0)),
        out_shape=jax.ShapeDtypeStruct((L, B, D), jnp.float32),
        scratch_shapes=[pltpu.VMEM((RB, D), jnp.bfloat16)],
        input_output_aliases=aliases,
    )(*args)


def _permute_idx(x):
    return (jnp.swapaxes(x.reshape(NB, BB, L), 1, 2)
            .reshape(N).astype(jnp.int32))


def kernel(vid, aid, tag, ts, playtime, dur, label, vid_table, aid_table,
           tag_W, tag_b, ts_W, ts_b, pt_W, pt_b, dur_W, dur_b, label_W,
           label_b, p1_W, p1_b, p2_W, p2_b):
    vid_p = _permute_idx(vid)
    aid_p = _permute_idx(aid)
    wsf, b1 = _fold(tag_W, ts_W, pt_W, dur_W, label_W, tag_b, ts_b, pt_b,
                    dur_b, label_b, p1_W, p1_b)
    tag_t = jnp.transpose(tag, (1, 2, 0))
    lab_t = jnp.transpose(label, (2, 1, 0))
    ts_t = ts.T
    pt_t = playtime.T
    dur_t = dur.T
    gathered = [
        _sc_gather(lax.slice(vid_p, (GSTART[g] * RB,),
                             ((GSTART[g] + GSIZES[g]) * RB,)),
                   lax.slice(aid_p, (GSTART[g] * RB,),
                             ((GSTART[g] + GSIZES[g]) * RB,)),
                   vid_table, aid_table)
        for g in range(G)
    ]
    out = None
    for g in range(G):
        gv, ga = gathered[g]
        out = _mlp_group(g, out, gv, ga, tag_t, lab_t, ts_t, pt_t, dur_t,
                         p1_W, wsf, b1, p2_W, p2_b)
    return jnp.transpose(out, (1, 0, 2))


# speedup vs baseline: 1.0390x; 1.0213x over previous
import jax
import jax.numpy as jnp
from jax import lax
from jax.experimental import pallas as pl
from jax.experimental.pallas import tpu as pltpu
from jax.experimental.pallas import tpu_sc as plsc

B, L, V = 4096, 20, 100000
N = B * L
D = 512
SFDIM = 113

G = 4
NG = N // G

NC, NS = 2, 16
NW = NC * NS
PER_W = NG // NW
CHUNK = 32
NBODY = PER_W // CHUNK // 2


def _gather_body(vid_ref, aid_ref, vtab_ref, atab_ref, gv_ref, ga_ref,
                 ia0, ib0, ia1, ib1, ra0, rb0, ra1, rb1,
                 sga0, sgb0, sga1, sgb1, soa0, sob0, soa1, sob1):
    wid = lax.axis_index("s") * NC + lax.axis_index("c")
    base = wid * PER_W

    slots = (
        (vid_ref, vtab_ref, gv_ref, ia0, ra0, sga0, soa0, 0),
        (aid_ref, atab_ref, ga_ref, ib0, rb0, sgb0, sob0, 0),
        (vid_ref, vtab_ref, gv_ref, ia1, ra1, sga1, soa1, 1),
        (aid_ref, atab_ref, ga_ref, ib1, rb1, sgb1, sob1, 1),
    )

    def body(cc, carry):
        gathers = []
        for (src_idx, table, dst, idx, rows, sg, so, par) in slots:
            off = base + (2 * cc + par) * CHUNK

            @pl.when(cc > 0)
            def _():
                pltpu.make_async_copy(dst.at[pl.ds(0, CHUNK)], rows,
                                      so).wait()

            pltpu.sync_copy(src_idx.at[pl.ds(off, CHUNK)], idx)
            gathers.append(pltpu.async_copy(table.at[idx], rows, sg))
        for (src_idx, table, dst, idx, rows, sg, so, par), g in zip(slots,
                                                                    gathers):
            off = base + (2 * cc + par) * CHUNK
            g.wait()
            pltpu.async_copy(rows, dst.at[pl.ds(off, CHUNK)], so)
        return carry

    lax.fori_loop(0, NBODY, body, 0)
    for (src_idx, table, dst, idx, rows, sg, so, par) in slots:
        pltpu.make_async_copy(dst.at[pl.ds(0, CHUNK)], rows, so).wait()


def _sc_gather(vid_flat, aid_flat, vid_table, aid_table):
    mesh = plsc.VectorSubcoreMesh(core_axis_name="c", subcore_axis_name="s",
                                  num_cores=NC, num_subcores=NS)
    return pl.kernel(
        _gather_body,
        out_type=(jax.ShapeDtypeStruct((NG, D), jnp.float32),
                  jax.ShapeDtypeStruct((NG, D), jnp.float32)),
        mesh=mesh,
        scratch_types=[
            pltpu.VMEM((CHUNK,), jnp.int32),
            pltpu.VMEM((CHUNK,), jnp.int32),
            pltpu.VMEM((CHUNK,), jnp.int32),
            pltpu.VMEM((CHUNK,), jnp.int32),
            pltpu.VMEM((CHUNK, D), jnp.float32),
            pltpu.VMEM((CHUNK, D), jnp.float32),
            pltpu.VMEM((CHUNK, D), jnp.float32),
            pltpu.VMEM((CHUNK, D), jnp.float32),
            pltpu.SemaphoreType.DMA,
            pltpu.SemaphoreType.DMA,
            pltpu.SemaphoreType.DMA,
            pltpu.SemaphoreType.DMA,
            pltpu.SemaphoreType.DMA,
            pltpu.SemaphoreType.DMA,
            pltpu.SemaphoreType.DMA,
            pltpu.SemaphoreType.DMA,
        ],
    )(vid_flat, aid_flat, vid_table, aid_table)


def _fold_body(tagW, tsW, ptW, durW, labelW, tagb, tsb, ptb, durb, labelb,
               p1W, p1b, wsf_ref, bias_ref):
    Wt = p1W[1024:1152, :]
    Wts = p1W[1152:1280, :]
    Wp = p1W[1280:1408, :]
    Wd = p1W[1408:1536, :]
    Wl = p1W[1536:1664, :]
    f32 = jnp.float32
    wsf_ref[0:100, :] = jnp.dot(tagW[...], Wt, preferred_element_type=f32)
    wsf_ref[100:101, :] = jnp.dot(tsW[...], Wts, preferred_element_type=f32)
    wsf_ref[101:102, :] = jnp.dot(ptW[...], Wp, preferred_element_type=f32)
    wsf_ref[102:103, :] = jnp.dot(durW[...], Wd, preferred_element_type=f32)
    wsf_ref[103:113, :] = jnp.dot(labelW[...], Wl, preferred_element_type=f32)
    bias_ref[...] = (
        jnp.dot(tagb[...], Wt, preferred_element_type=f32)
        + jnp.dot(tsb[...], Wts, preferred_element_type=f32)
        + jnp.dot(ptb[...], Wp, preferred_element_type=f32)
        + jnp.dot(durb[...], Wd, preferred_element_type=f32)
        + jnp.dot(labelb[...], Wl, preferred_element_type=f32)
        + p1b[...]
    )


def _fold(tag_W, ts_W, pt_W, dur_W, label_W, tag_b, ts_b, pt_b, dur_b,
          label_b, p1_W, p1_b):
    row = lambda b: b.reshape(1, -1)
    return pl.pallas_call(
        _fold_body,
        out_shape=(jax.ShapeDtypeStruct((SFDIM, D), jnp.float32),
                   jax.ShapeDtypeStruct((1, D), jnp.float32)),
    )(tag_W, ts_W, pt_W, dur_W, label_W, row(tag_b), row(ts_b), row(pt_b),
      row(dur_b), row(label_b), p1_W, row(p1_b))


BB = 128
NB = B // BB
RB = BB * L

_DNT = (((0,), (0,)), ((), ()))


def _mlp_body(gv, ga, tag_r, lab_r, ts_r, pt_r, dur_r, Wv, Wa, Wsf, b1,
              p2W, p2b, out, h_scr):
    f32, bf16 = jnp.float32, jnp.bfloat16
    bigh = jnp.dot(gv[...].astype(bf16), Wv[...].astype(bf16),
                   preferred_element_type=f32)
    bigh = bigh + jnp.dot(ga[...].astype(bf16), Wa[...].astype(bf16),
                          preferred_element_type=f32)
    wsf_b = Wsf[...].astype(bf16)
    tag_b16 = tag_r[...].astype(bf16)
    lab_b16 = lab_r[...].astype(bf16)
    for l in range(L):
        hl = bigh[l * BB:(l + 1) * BB, :]
        hl = hl + lax.dot_general(tag_b16[l], wsf_b[0:100, :], _DNT,
                                  preferred_element_type=f32)
        hl = hl + lax.dot_general(lab_b16[:, l, :], wsf_b[103:113, :], _DNT,
                                  preferred_element_type=f32)
        s3 = jnp.concatenate([ts_r[l:l + 1, :], pt_r[l:l + 1, :],
                              dur_r[l:l + 1, :]], axis=0)
        hl = hl + lax.dot_general(s3, Wsf[100:103, :], _DNT,
                                  preferred_element_type=f32)
        hl = hl + b1[...]
        hl = jnp.where(hl >= 0, hl, 0.01 * hl)
        h_scr[l * BB:(l + 1) * BB, :] = hl.astype(bf16)
    out2 = jnp.dot(h_scr[...], p2W[...].astype(bf16),
                   preferred_element_type=f32)
    out2 = out2 + p2b[...]
    for l in range(L):
        out[l, :, :] = out2[l * BB:(l + 1) * BB, :]


def _mlp_body_aliased(prev, *args):
    del prev
    _mlp_body(*args)


NBG = NB // G


def _mlp_group(g, out_prev, gv, ga, tag, label, ts, playtime, dur, p1_W,
               wsf, b1, p2_W, p2_b):
    const = lambda shape: pl.BlockSpec(shape, lambda i: (0, 0))
    o = g * NBG
    in_specs = [
        pl.BlockSpec((RB, D), lambda i: (i, 0)),
        pl.BlockSpec((RB, D), lambda i: (i, 0)),
        pl.BlockSpec((L, 100, BB), lambda i: (0, 0, o + i)),
        pl.BlockSpec((10, L, BB), lambda i: (0, 0, o + i)),
        pl.BlockSpec((L, BB), lambda i: (0, o + i)),
        pl.BlockSpec((L, BB), lambda i: (0, o + i)),
        pl.BlockSpec((L, BB), lambda i: (0, o + i)),
        pl.BlockSpec((D, D), lambda i: (0, 0)),
        pl.BlockSpec((D, D), lambda i: (1, 0)),
        const((SFDIM, D)),
        const((1, D)),
        const((D, D)),
        const((1, D)),
    ]
    args = (gv, ga, tag, label, ts, playtime, dur, p1_W, p1_W, wsf, b1,
            p2_W, p2_b.reshape(1, D))
    body = _mlp_body
    aliases = {}
    if out_prev is not None:
        in_specs = [pl.BlockSpec(memory_space=pl.ANY)] + in_specs
        args = (out_prev,) + args
        body = _mlp_body_aliased
        aliases = {0: 0}
    return pl.pallas_call(
        body,
        grid=(NBG,),
        in_specs=in_specs,
        out_specs=pl.BlockSpec((L, BB, D), lambda i: (0, o + i, 0)),
        out_shape=jax.ShapeDtypeStruct((L, B, D), jnp.float32),
        scratch_shapes=[pltpu.VMEM((RB, D), jnp.bfloat16)],
        input_output_aliases=aliases,
    )(*args)


def _permute_idx(x):
    return (jnp.swapaxes(x.reshape(NB, BB, L), 1, 2)
            .reshape(N).astype(jnp.int32))


def kernel(vid, aid, tag, ts, playtime, dur, label, vid_table, aid_table,
           tag_W, tag_b, ts_W, ts_b, pt_W, pt_b, dur_W, dur_b, label_W,
           label_b, p1_W, p1_b, p2_W, p2_b):
    vid_p = _permute_idx(vid)
    aid_p = _permute_idx(aid)
    wsf, b1 = _fold(tag_W, ts_W, pt_W, dur_W, label_W, tag_b, ts_b, pt_b,
                    dur_b, label_b, p1_W, p1_b)
    tag_t = jnp.transpose(tag, (1, 2, 0))
    lab_t = jnp.transpose(label, (2, 1, 0))
    ts_t = ts.T
    pt_t = playtime.T
    dur_t = dur.T
    gathered = [
        _sc_gather(lax.slice(vid_p, (g * NG,), ((g + 1) * NG,)),
                   lax.slice(aid_p, (g * NG,), ((g + 1) * NG,)),
                   vid_table, aid_table)
        for g in range(G)
    ]
    out = None
    for g in range(G):
        gv, ga = gathered[g]
        out = _mlp_group(g, out, gv, ga, tag_t, lab_t, ts_t, pt_t, dur_t,
                         p1_W, wsf, b1, p2_W, p2_b)
    return jnp.transpose(out, (1, 0, 2))
